# Initial kernel scaffold; baseline (speedup 1.0000x reference)
import jax
import jax.numpy as jnp
from jax.experimental import pallas as pl


def kernel(x, embed_weight):
    raise NotImplementedError("write your pallas kernel here")


# speedup vs baseline: 1.5719x; 1.5719x over previous
import functools

import jax
import jax.numpy as jnp
from jax import lax
from jax.experimental import pallas as pl
from jax.experimental.pallas import tpu as pltpu
from jax.experimental.pallas import tpu_sc as plsc

NC = 2
NS = 16
NW = NC * NS

W = 32


@functools.partial(jax.jit, static_argnames=())
def _sc_gather(table, idx):
    b_total = idx.shape[0]
    d = table.shape[1]
    assert b_total % NW == 0
    b_per_w = b_total // NW
    assert b_per_w % W == 0
    n_win = b_per_w // W

    mesh = plsc.VectorSubcoreMesh(core_axis_name="c", subcore_axis_name="s")

    @functools.partial(
        pl.kernel,
        mesh=mesh,
        out_type=jax.ShapeDtypeStruct((b_total, d), table.dtype),
        scratch_types=[
            pltpu.VMEM((b_per_w,), jnp.int32),
            pltpu.VMEM((W, d), table.dtype),
            pltpu.SemaphoreType.DMA,
        ],
    )
    def k(table_hbm, idx_hbm, out_hbm, idx_v, rows_v, sem):
        wid = lax.axis_index("s") * NC + lax.axis_index("c")
        base = wid * b_per_w
        pltpu.sync_copy(idx_hbm.at[pl.ds(base, b_per_w)], idx_v)

        @pl.loop(0, n_win)
        def _(i):
            off = i * W
            pltpu.async_copy(
                table_hbm.at[idx_v.at[pl.ds(off, W)]], rows_v, sem
            ).wait()
            pltpu.sync_copy(rows_v, out_hbm.at[pl.ds(base + off, W)])

    return k(table, idx)


def kernel(x, embed_weight):
    b, t = x.shape
    d = embed_weight.shape[1]
    flat = x.reshape(b * t).astype(jnp.int32)
    out = _sc_gather(embed_weight, flat)
    return out.reshape(b, t, d)


# speedup vs baseline: 1.7795x; 1.1321x over previous
import functools

import jax
import jax.numpy as jnp
from jax import lax
from jax.experimental import pallas as pl
from jax.experimental.pallas import tpu as pltpu
from jax.experimental.pallas import tpu_sc as plsc

NC = 2
NS = 16
NW = NC * NS

W = 64


def _sc_gather(table, idx):
    b_total = idx.shape[0]
    d = table.shape[1]
    assert b_total % NW == 0
    b_per_w = b_total // NW
    n_full = b_per_w // W
    tail = b_per_w - n_full * W
    n_pairs = n_full // 2
    assert n_full % 2 == 0 and tail % 8 == 0

    mesh = plsc.VectorSubcoreMesh(core_axis_name="c", subcore_axis_name="s")

    @functools.partial(
        pl.kernel,
        mesh=mesh,
        out_type=jax.ShapeDtypeStruct((b_total, d), table.dtype),
        scratch_types=[
            pltpu.VMEM((b_per_w,), jnp.int32),
            pltpu.VMEM((W, d), table.dtype),
            pltpu.VMEM((W, d), table.dtype),
            pltpu.SemaphoreType.DMA,
            pltpu.SemaphoreType.DMA,
            pltpu.SemaphoreType.DMA,
            pltpu.SemaphoreType.DMA,
        ],
    )
    def k(table_hbm, idx_hbm, out_hbm, idx_v, rows0, rows1, gs0, gs1, os0, os1):
        wid = lax.axis_index("s") * NC + lax.axis_index("c")
        base = wid * b_per_w
        pltpu.sync_copy(idx_hbm.at[pl.ds(base, b_per_w)], idx_v)

        def gather(win, buf, sem):
            return pltpu.make_async_copy(
                table_hbm.at[idx_v.at[pl.ds(win * W, W)]], buf, sem)

        def drain(win, buf, sem):
            return pltpu.make_async_copy(
                buf, out_hbm.at[pl.ds(base + win * W, W)], sem)

        gather(0, rows0, gs0).start()
        gather(1, rows1, gs1).start()

        @pl.loop(0, n_pairs)
        def _(i):
            w0 = 2 * i
            w1 = w0 + 1
            gather(w0, rows0, gs0).wait()
            drain(w0, rows0, os0).start()
            gather(w1, rows1, gs1).wait()
            drain(w1, rows1, os1).start()

            @pl.when(i < n_pairs - 1)
            def _():
                drain(w0, rows0, os0).wait()
                gather(w0 + 2, rows0, gs0).start()
                drain(w1, rows1, os1).wait()
                gather(w1 + 2, rows1, gs1).start()

        drain(0, rows0, os0).wait()
        drain(0, rows1, os1).wait()

        if tail:
            t_off = n_full * W
            tbuf = rows0.at[pl.ds(0, tail)]
            pltpu.async_copy(
                table_hbm.at[idx_v.at[pl.ds(t_off, tail)]], tbuf, gs0).wait()
            pltpu.sync_copy(tbuf, out_hbm.at[pl.ds(base + t_off, tail)])

    return k(table, idx)


def kernel(x, embed_weight):
    b, t = x.shape
    d = embed_weight.shape[1]
    flat = x.reshape(b * t).astype(jnp.int32)
    out = _sc_gather(embed_weight, flat)
    return out.reshape(b, t, d)


# speedup vs baseline: 6.2315x; 3.5018x over previous
import functools

import jax
import jax.numpy as jnp
from jax import lax
from jax.experimental import pallas as pl
from jax.experimental.pallas import tpu as pltpu
from jax.experimental.pallas import tpu_sc as plsc

NC = 2
NS = 16
NW = NC * NS

W = 64


def _sc_gather(table, idx):
    b_total = idx.shape[0]
    d = table.shape[1]
    assert b_total % NW == 0
    b_per_w = b_total // NW
    n_full = b_per_w // W
    tail = b_per_w - n_full * W
    n_pairs = n_full // 2
    assert n_full % 2 == 0 and tail % 8 == 0

    mesh = plsc.VectorSubcoreMesh(core_axis_name="c", subcore_axis_name="s")

    @functools.partial(
        pl.kernel,
        mesh=mesh,
        out_type=jax.ShapeDtypeStruct((b_total, d), table.dtype),
        scratch_types=[
            pltpu.VMEM((b_per_w,), jnp.int32),
            pltpu.VMEM((W, d), table.dtype),
            pltpu.VMEM((W, d), table.dtype),
            pltpu.SemaphoreType.DMA,
            pltpu.SemaphoreType.DMA,
            pltpu.SemaphoreType.DMA,
            pltpu.SemaphoreType.DMA,
        ],
    )
    def k(table_hbm, idx_hbm, out_hbm, idx_v, rows0, rows1, gs0, gs1, os0, os1):
        wid = lax.axis_index("s") * NC + lax.axis_index("c")
        base = wid * b_per_w
        pltpu.sync_copy(idx_hbm.at[pl.ds(base, b_per_w)], idx_v)

        def gather(win, buf, sem):
            return pltpu.make_async_copy(
                table_hbm.at[idx_v.at[pl.ds(win * W, W)]], buf, sem)

        def drain(win, buf, sem):
            return pltpu.make_async_copy(
                buf, out_hbm.at[pl.ds(base + win * W, W)], sem)

        gather(0, rows0, gs0).start()
        gather(1, rows1, gs1).start()

        @pl.loop(0, n_pairs)
        def _(i):
            w0 = 2 * i
            w1 = w0 + 1
            gather(w0, rows0, gs0).wait()
            drain(w0, rows0, os0).start()
            gather(w1, rows1, gs1).wait()
            drain(w1, rows1, os1).start()

            @pl.when(i < n_pairs - 1)
            def _():
                drain(w0, rows0, os0).wait()
                gather(w0 + 2, rows0, gs0).start()
                drain(w1, rows1, os1).wait()
                gather(w1 + 2, rows1, gs1).start()

        drain(0, rows0, os0).wait()
        drain(0, rows1, os1).wait()

        if tail:
            t_off = n_full * W
            tbuf = rows0.at[pl.ds(0, tail)]
            pltpu.async_copy(
                table_hbm.at[idx_v.at[pl.ds(t_off, tail)]], tbuf, gs0).wait()
            pltpu.sync_copy(tbuf, out_hbm.at[pl.ds(base + t_off, tail)])

    return k(table, idx)


def kernel(x, embed_weight):
    b, t = x.shape
    d = embed_weight.shape[1]
    flat = x.T.reshape(b * t).astype(jnp.int32)
    out = _sc_gather(embed_weight, flat)
    return out.reshape(t, b, d).transpose(1, 0, 2)


# speedup vs baseline: 6.2507x; 1.0031x over previous
import functools

import jax
import jax.numpy as jnp
from jax import lax
from jax.experimental import pallas as pl
from jax.experimental.pallas import tpu as pltpu
from jax.experimental.pallas import tpu_sc as plsc

NC = 2
NS = 16
NW = NC * NS

W = 72
NBUF = 3


def _sc_gather(table, idx):
    b_total = idx.shape[0]
    d = table.shape[1]
    assert b_total % NW == 0
    b_per_w = b_total // NW
    n_full = b_per_w // W
    tail = b_per_w - n_full * W
    n_rounds = n_full // NBUF
    assert n_full % NBUF == 0 and tail % 8 == 0

    mesh = plsc.VectorSubcoreMesh(core_axis_name="c", subcore_axis_name="s")

    @functools.partial(
        pl.kernel,
        mesh=mesh,
        out_type=jax.ShapeDtypeStruct((b_total, d), table.dtype),
        scratch_types=[
            pltpu.VMEM((b_per_w,), jnp.int32),
        ] + [pltpu.VMEM((W, d), table.dtype) for _ in range(NBUF)]
          + [pltpu.SemaphoreType.DMA for _ in range(2 * NBUF)],
    )
    def k(table_hbm, idx_hbm, out_hbm, idx_v, *bufs_and_sems):
        bufs = bufs_and_sems[:NBUF]
        gs = bufs_and_sems[NBUF:2 * NBUF]
        os = bufs_and_sems[2 * NBUF:]
        wid = lax.axis_index("s") * NC + lax.axis_index("c")
        base = wid * b_per_w
        pltpu.sync_copy(idx_hbm.at[pl.ds(base, b_per_w)], idx_v)

        def gather(win, b):
            return pltpu.make_async_copy(
                table_hbm.at[idx_v.at[pl.ds(win * W, W)]], bufs[b], gs[b])

        def drain(win, b):
            return pltpu.make_async_copy(
                bufs[b], out_hbm.at[pl.ds(base + win * W, W)], os[b])

        for b in range(NBUF):
            gather(b, b).start()

        @pl.loop(0, n_rounds)
        def _(r):
            for b in range(NBUF):
                w = NBUF * r + b
                gather(w, b).wait()
                drain(w, b).start()

                @pl.when(r < n_rounds - 1)
                def _():
                    drain(w, b).wait()
                    gather(w + NBUF, b).start()

        for b in range(NBUF):
            drain(0, b).wait()

        if tail:
            t_off = n_full * W
            tbuf = bufs[0].at[pl.ds(0, tail)]
            pltpu.async_copy(
                table_hbm.at[idx_v.at[pl.ds(t_off, tail)]], tbuf, gs[0]).wait()
            pltpu.sync_copy(tbuf, out_hbm.at[pl.ds(base + t_off, tail)])

    return k(table, idx)


def kernel(x, embed_weight):
    b, t = x.shape
    d = embed_weight.shape[1]
    flat = x.T.reshape(b * t).astype(jnp.int32)
    out = _sc_gather(embed_weight, flat)
    return out.reshape(t, b, d).transpose(1, 0, 2)
